# baseline (speedup 1.0000x reference)
import functools

import jax
import jax.numpy as jnp
from jax import lax
from jax.experimental import pallas as pl
from jax.experimental.pallas import tpu as pltpu
from jax.experimental.pallas import tpu_sc as plsc

_EMB = 1000000
_D = 64
_B = 16384
_K = 5

_NC = 2
_NS = 16
_NW = _NC * _NS
_BPW = _B // _NW
_CH = 128
_UCH = _BPW // _CH

_mesh = plsc.VectorSubcoreMesh(core_axis_name="c", subcore_axis_name="s")


_TR = 512
_TG = -(-_EMB // (2 * _TR))
_L = _TG * _TR


def _transpose_body(xl_ref, xr_ref, o_ref):
    xl = xl_ref[...]
    xr = xr_ref[...]
    eye = jnp.float32(
        lax.broadcasted_iota(jnp.int32, (_D, _D), 0)
        == lax.broadcasted_iota(jnp.int32, (_D, _D), 1))
    dn = (((0,), (0,)), ((), ()))
    yl = lax.dot_general(xl, eye, dn, preferred_element_type=jnp.float32)
    yr = lax.dot_general(xr, eye, dn, preferred_element_type=jnp.float32)
    o_ref[...] = jnp.concatenate([yl, yr], axis=1)


_pack_pairs = pl.pallas_call(
    _transpose_body,
    grid=(_TG,),
    in_specs=[
        pl.BlockSpec((_D, _TR), lambda i: (0, i)),
        pl.BlockSpec((_D, _TR), lambda i: (0, i + _TG)),
    ],
    out_specs=pl.BlockSpec((_TR, 2 * _D), lambda i: (i, 0)),
    out_shape=jax.ShapeDtypeStruct((_L, 2 * _D), jnp.float32),
)


@functools.partial(
    pl.kernel,
    mesh=_mesh,
    out_type=[
        jax.ShapeDtypeStruct((_B, 2 * _D), jnp.float32),
        jax.ShapeDtypeStruct((_B, 2 * _D), jnp.float32),
        jax.ShapeDtypeStruct((_K * _B, 2 * _D), jnp.float32),
    ],
    scratch_types=[
        pltpu.VMEM((_BPW,), jnp.int32),
        pltpu.VMEM((_BPW,), jnp.int32),
        pltpu.VMEM((_K, _BPW), jnp.int32),
        pltpu.VMEM((2, _CH, 2 * _D), jnp.float32),
        pltpu.SemaphoreType.DMA,
        pltpu.SemaphoreType.DMA,
    ],
)
def _sc_gather(pos_u, pos_v, neg_vt, u_p, v_p,
               out_u, out_v, out_n,
               idx_u, idx_v, idx_n, rows, sem0, sem1):
    c = lax.axis_index("c")
    s = lax.axis_index("s")
    wid = s * _NC + c
    base = wid * _BPW

    pltpu.sync_copy(pos_u.at[pl.ds(base, _BPW)], idx_u)
    pltpu.sync_copy(pos_v.at[pl.ds(base, _BPW)], idx_v)
    pltpu.sync_copy(neg_vt.at[:, pl.ds(base, _BPW)], idx_n)

    sems = (sem0, sem1)

    def chunk_seq(table, vec_of, nch, out, obase):
        def fire(j, slot):
            def body(g, carry):
                v0 = vec_of(j, g)
                vec = jnp.where(v0 >= _L, v0 - _L, v0)
                for k in range(16):
                    pltpu.async_copy(
                        table.at[vec[k]], rows.at[slot].at[g * 16 + k],
                        sems[slot])
                return carry
            lax.fori_loop(0, _CH // 16, body, 0)

        def drain(slot):
            pltpu.make_async_copy(
                out.at[pl.ds(0, _CH)], rows.at[slot], sems[slot]).wait()

        fire(0, 0)
        for j in range(nch):
            slot = j % 2
            if j + 1 < nch:
                fire(j + 1, 1 - slot)
            drain(slot)
            pltpu.sync_copy(rows.at[slot], out.at[pl.ds(obase + j * _CH, _CH)])

    chunk_seq(u_p, lambda j, g: idx_u[pl.ds(j * _CH + g * 16, 16)],
              _UCH, out_u, base)
    chunk_seq(v_p, lambda j, g: idx_v[pl.ds(j * _CH + g * 16, 16)],
              _UCH, out_v, base)
    for k in range(_K):
        chunk_seq(v_p,
                  lambda j, g, _k=k: idx_n[_k, pl.ds(j * _CH + g * 16, 16)],
                  _UCH, out_n, k * _B + base)


_BLK = 1024
_G = _B // _BLK


def _half(pair, idx):
    lo = pair[..., :_D]
    hi = pair[..., _D:]
    return jnp.where(idx[..., None] >= _L, hi, lo)


def _tc_loss_body(pu_ref, pv_ref, nv_ref, u_ref, v_ref, n_ref, out_ref):
    u = _half(u_ref[...], pu_ref[...])
    v = _half(v_ref[...], pv_ref[...])
    n = _half(n_ref[...], nv_ref[...])
    score = jnp.sum(u * v, axis=1)
    score = jnp.clip(score, -10.0, 10.0)
    pos_l = jnp.log1p(jnp.exp(-score))
    ns = jnp.sum(n * u[None, :, :], axis=-1)
    ns = jnp.clip(ns, -10.0, 10.0)
    neg_l = jnp.sum(jnp.log1p(jnp.exp(ns)), axis=0)
    inc = (jnp.sum(pos_l + neg_l) * (1.0 / _B))[None, None]

    @pl.when(pl.program_id(0) == 0)
    def _():
        out_ref[...] = jnp.zeros((1, 1), jnp.float32)

    out_ref[...] += inc


_tc_loss = pl.pallas_call(
    _tc_loss_body,
    grid=(_G,),
    in_specs=[
        pl.BlockSpec((_BLK,), lambda i: (i,)),
        pl.BlockSpec((_BLK,), lambda i: (i,)),
        pl.BlockSpec((_K, _BLK), lambda i: (0, i)),
        pl.BlockSpec((_BLK, 2 * _D), lambda i: (i, 0)),
        pl.BlockSpec((_BLK, 2 * _D), lambda i: (i, 0)),
        pl.BlockSpec((_K, _BLK, 2 * _D), lambda i: (0, i, 0)),
    ],
    out_specs=pl.BlockSpec((1, 1), lambda i: (0, 0)),
    out_shape=jax.ShapeDtypeStruct((1, 1), jnp.float32),
)


def kernel(pos_u, pos_v, neg_v, u_embs, v_embs):
    pos_u = pos_u.astype(jnp.int32)
    pos_v = pos_v.astype(jnp.int32)
    neg_vt = neg_v.T.astype(jnp.int32)
    u_t = u_embs.T
    v_t = v_embs.T
    u_p = _pack_pairs(u_t, u_t)
    v_p = _pack_pairs(v_t, v_t)
    rows_u, rows_v, rows_n = _sc_gather(pos_u, pos_v, neg_vt, u_p, v_p)
    out = _tc_loss(pos_u, pos_v, neg_vt, rows_u, rows_v,
                   rows_n.reshape(_K, _B, 2 * _D))
    return out[0, 0]


# speedup vs baseline: 2.0913x; 2.0913x over previous
import functools

import jax
import jax.numpy as jnp
from jax import lax
from jax.experimental import pallas as pl
from jax.experimental.pallas import tpu as pltpu
from jax.experimental.pallas import tpu_sc as plsc

_EMB = 1000000
_D = 64
_B = 16384
_K = 5

_NC = 2
_NS = 16
_NW = _NC * _NS
_BPW = _B // _NW
_CH = 128
_UCH = _BPW // _CH

_mesh = plsc.VectorSubcoreMesh(core_axis_name="c", subcore_axis_name="s")


def _worker_base():
    c = lax.axis_index("c")
    s = lax.axis_index("s")
    return (s * _NC + c) * _BPW


def _chunk_seq(table, idx_of, nch, out, obase, rows, sems):
    def fire(j, slot):
        def body(g, carry):
            vec = idx_of(j, g)
            for k in range(16):
                pltpu.async_copy(
                    table.at[vec[k]], rows.at[slot].at[g * 16 + k],
                    sems[slot])
            return carry
        lax.fori_loop(0, _CH // 16, body, 0)

    def drain(slot):
        pltpu.make_async_copy(
            out.at[pl.ds(0, _CH)], rows.at[slot], sems[slot]).wait()

    fire(0, 0)
    for j in range(nch):
        slot = j % 2
        if j + 1 < nch:
            fire(j + 1, 1 - slot)
        drain(slot)
        pltpu.sync_copy(rows.at[slot], out.at[pl.ds(obase + j * _CH, _CH)])


@functools.partial(
    pl.kernel,
    mesh=_mesh,
    out_type=[jax.ShapeDtypeStruct((_B, _D), jnp.float32)],
    scratch_types=[
        pltpu.VMEM((_BPW,), jnp.int32),
        pltpu.VMEM((2, _CH, _D), jnp.float32),
        pltpu.SemaphoreType.DMA,
        pltpu.SemaphoreType.DMA,
    ],
)
def _sc_gather_u(pos_u, u_embs, out_u, idx_u, rows, sem0, sem1):
    base = _worker_base()
    pltpu.sync_copy(pos_u.at[pl.ds(base, _BPW)], idx_u)
    _chunk_seq(u_embs, lambda j, g: idx_u[pl.ds(j * _CH + g * 16, 16)],
               _UCH, out_u, base, rows, (sem0, sem1))


@functools.partial(
    pl.kernel,
    mesh=_mesh,
    out_type=[
        jax.ShapeDtypeStruct((_B, _D), jnp.float32),
        jax.ShapeDtypeStruct((_K * _B, _D), jnp.float32),
    ],
    scratch_types=[
        pltpu.VMEM((_BPW,), jnp.int32),
        pltpu.VMEM((_K, _BPW), jnp.int32),
        pltpu.VMEM((2, _CH, _D), jnp.float32),
        pltpu.SemaphoreType.DMA,
        pltpu.SemaphoreType.DMA,
    ],
)
def _sc_gather_vn(pos_v, neg_vt, v_embs, out_v, out_n,
                  idx_v, idx_n, rows, sem0, sem1):
    base = _worker_base()
    sems = (sem0, sem1)
    pltpu.sync_copy(pos_v.at[pl.ds(base, _BPW)], idx_v)
    pltpu.sync_copy(neg_vt.at[:, pl.ds(base, _BPW)], idx_n)
    _chunk_seq(v_embs, lambda j, g: idx_v[pl.ds(j * _CH + g * 16, 16)],
               _UCH, out_v, base, rows, sems)
    for k in range(_K):
        _chunk_seq(v_embs,
                   lambda j, g, _k=k: idx_n[_k, pl.ds(j * _CH + g * 16, 16)],
                   _UCH, out_n, k * _B + base, rows, sems)


_BLK = 1024
_G = _B // _BLK


def _tc_loss_body(u_ref, v_ref, n_ref, out_ref):
    u = u_ref[...]
    v = v_ref[...]
    n = n_ref[...]
    score = jnp.sum(u * v, axis=1)
    score = jnp.clip(score, -10.0, 10.0)
    pos_l = jnp.log1p(jnp.exp(-score))
    ns = jnp.sum(n * u[None, :, :], axis=-1)
    ns = jnp.clip(ns, -10.0, 10.0)
    neg_l = jnp.sum(jnp.log1p(jnp.exp(ns)), axis=0)
    inc = (jnp.sum(pos_l + neg_l) * (1.0 / _B))[None, None]

    @pl.when(pl.program_id(0) == 0)
    def _():
        out_ref[...] = jnp.zeros((1, 1), jnp.float32)

    out_ref[...] += inc


_tc_loss = pl.pallas_call(
    _tc_loss_body,
    grid=(_G,),
    in_specs=[
        pl.BlockSpec((_BLK, _D), lambda i: (i, 0)),
        pl.BlockSpec((_BLK, _D), lambda i: (i, 0)),
        pl.BlockSpec((_K, _BLK, _D), lambda i: (0, i, 0)),
    ],
    out_specs=pl.BlockSpec((1, 1), lambda i: (0, 0)),
    out_shape=jax.ShapeDtypeStruct((1, 1), jnp.float32),
)


def kernel(pos_u, pos_v, neg_v, u_embs, v_embs):
    (rows_u,) = _sc_gather_u(pos_u.astype(jnp.int32), u_embs)
    rows_v, rows_n = _sc_gather_vn(
        pos_v.astype(jnp.int32), neg_v.T.astype(jnp.int32), v_embs)
    out = _tc_loss(rows_u, rows_v, rows_n.reshape(_K, _B, _D))
    return out[0, 0]


# speedup vs baseline: 2.0933x; 1.0009x over previous
import functools

import jax
import jax.numpy as jnp
from jax import lax
from jax.experimental import pallas as pl
from jax.experimental.pallas import tpu as pltpu
from jax.experimental.pallas import tpu_sc as plsc

_EMB = 1000000
_D = 64
_B = 16384
_K = 5

_NC = 2
_NS = 16
_NW = _NC * _NS
_BPW = _B // _NW
_CH = 256
_UCH = _BPW // _CH

_mesh = plsc.VectorSubcoreMesh(core_axis_name="c", subcore_axis_name="s")


def _worker_base():
    c = lax.axis_index("c")
    s = lax.axis_index("s")
    return (s * _NC + c) * _BPW


def _chunk_seq(table, idx_of, nch, out, obase, rows, sems):
    def fire(j, slot):
        def body(g, carry):
            vec = idx_of(j, g)
            for k in range(16):
                pltpu.async_copy(
                    table.at[vec[k]], rows.at[slot].at[g * 16 + k],
                    sems[slot])
            return carry
        lax.fori_loop(0, _CH // 16, body, 0)

    def drain(slot):
        pltpu.make_async_copy(
            out.at[pl.ds(0, _CH)], rows.at[slot], sems[slot]).wait()

    fire(0, 0)
    for j in range(nch):
        slot = j % 2
        if j + 1 < nch:
            fire(j + 1, 1 - slot)
        drain(slot)
        pltpu.sync_copy(rows.at[slot], out.at[pl.ds(obase + j * _CH, _CH)])


@functools.partial(
    pl.kernel,
    mesh=_mesh,
    out_type=[jax.ShapeDtypeStruct((_B, _D), jnp.float32)],
    scratch_types=[
        pltpu.VMEM((_BPW,), jnp.int32),
        pltpu.VMEM((2, _CH, _D), jnp.float32),
        pltpu.SemaphoreType.DMA,
        pltpu.SemaphoreType.DMA,
    ],
)
def _sc_gather_u(pos_u, u_embs, out_u, idx_u, rows, sem0, sem1):
    base = _worker_base()
    pltpu.sync_copy(pos_u.at[pl.ds(base, _BPW)], idx_u)
    _chunk_seq(u_embs, lambda j, g: idx_u[pl.ds(j * _CH + g * 16, 16)],
               _UCH, out_u, base, rows, (sem0, sem1))


@functools.partial(
    pl.kernel,
    mesh=_mesh,
    out_type=[
        jax.ShapeDtypeStruct((_B, _D), jnp.float32),
        jax.ShapeDtypeStruct((_K * _B, _D), jnp.float32),
    ],
    scratch_types=[
        pltpu.VMEM((_BPW,), jnp.int32),
        pltpu.VMEM((_K, _BPW), jnp.int32),
        pltpu.VMEM((2, _CH, _D), jnp.float32),
        pltpu.SemaphoreType.DMA,
        pltpu.SemaphoreType.DMA,
    ],
)
def _sc_gather_vn(pos_v, neg_vt, v_embs, out_v, out_n,
                  idx_v, idx_n, rows, sem0, sem1):
    base = _worker_base()
    sems = (sem0, sem1)
    pltpu.sync_copy(pos_v.at[pl.ds(base, _BPW)], idx_v)
    pltpu.sync_copy(neg_vt.at[:, pl.ds(base, _BPW)], idx_n)
    _chunk_seq(v_embs, lambda j, g: idx_v[pl.ds(j * _CH + g * 16, 16)],
               _UCH, out_v, base, rows, sems)
    for k in range(_K):
        _chunk_seq(v_embs,
                   lambda j, g, _k=k: idx_n[_k, pl.ds(j * _CH + g * 16, 16)],
                   _UCH, out_n, k * _B + base, rows, sems)


_BLK = 1024
_G = _B // _BLK


def _tc_loss_body(u_ref, v_ref, n_ref, out_ref):
    u = u_ref[...]
    v = v_ref[...]
    n = n_ref[...]
    score = jnp.sum(u * v, axis=1)
    score = jnp.clip(score, -10.0, 10.0)
    pos_l = jnp.log1p(jnp.exp(-score))
    ns = jnp.sum(n * u[None, :, :], axis=-1)
    ns = jnp.clip(ns, -10.0, 10.0)
    neg_l = jnp.sum(jnp.log1p(jnp.exp(ns)), axis=0)
    inc = (jnp.sum(pos_l + neg_l) * (1.0 / _B))[None, None]

    @pl.when(pl.program_id(0) == 0)
    def _():
        out_ref[...] = jnp.zeros((1, 1), jnp.float32)

    out_ref[...] += inc


_tc_loss = pl.pallas_call(
    _tc_loss_body,
    grid=(_G,),
    in_specs=[
        pl.BlockSpec((_BLK, _D), lambda i: (i, 0)),
        pl.BlockSpec((_BLK, _D), lambda i: (i, 0)),
        pl.BlockSpec((_K, _BLK, _D), lambda i: (0, i, 0)),
    ],
    out_specs=pl.BlockSpec((1, 1), lambda i: (0, 0)),
    out_shape=jax.ShapeDtypeStruct((1, 1), jnp.float32),
)


def kernel(pos_u, pos_v, neg_v, u_embs, v_embs):
    (rows_u,) = _sc_gather_u(pos_u.astype(jnp.int32), u_embs)
    rows_v, rows_n = _sc_gather_vn(
        pos_v.astype(jnp.int32), neg_v.T.astype(jnp.int32), v_embs)
    out = _tc_loss(rows_u, rows_v, rows_n.reshape(_K, _B, _D))
    return out[0, 0]
